# baseline (device time: 46385 ns/iter reference)
import jax
import jax.numpy as jnp
from jax import lax
from jax.experimental import pallas as pl
from jax.experimental.pallas import tpu as pltpu

N_DEV = 4
M = 1024
D = 1024
HALF = M // 2
QUART = HALF // 2
EIGHT = HALF // 4


def kernel(partial, resid, gamma):
    x = partial
    g = gamma.reshape(1, D)

    def body(x_ref, resid_ref, g_ref, out_ref, r1, r2, r3, r4, t1, t2,
             xf1, xf2, xk1, xk2, rsd1, rsd2, gv, ov,
             send_sems, recv_sems, local_sems):
        p = lax.axis_index("i")
        q1 = p + 1 - 2 * (p % 2)
        q2 = 3 - p

        def xchg(idx, src, dst, partner):
            rdma = pltpu.make_async_remote_copy(
                src_ref=src, dst_ref=dst,
                send_sem=send_sems.at[idx], recv_sem=recv_sems.at[idx],
                device_id=(partner,), device_id_type=pl.DeviceIdType.MESH,
            )
            rdma.start()
            return rdma

        a1 = jnp.where((p == 0) | (p == 3), 0, QUART)
        c1 = jnp.where(p <= 1, 0, EIGHT)
        a2 = jnp.where(p <= 1, 0, QUART)
        c2 = jnp.where((p == 0) | (p == 2), 0, EIGHT)

        o1_ = a1 + c1
        o2_ = HALF + a2 + c2
        f1_ = EIGHT - c1
        f2_ = EIGHT - c2

        def lcopy(idx, src_r, dst_r):
            cp = pltpu.make_async_copy(src_r, dst_r, local_sems.at[idx])
            cp.start()
            return cp

        L0 = lcopy(0, x_ref.at[0, pl.ds(a1 + f1_, EIGHT), :], xf1)
        L1 = lcopy(1, x_ref.at[0, pl.ds(HALF + a2 + f2_, EIGHT), :], xf2)
        L2 = lcopy(2, x_ref.at[0, pl.ds(o1_, EIGHT), :], xk1)
        L3 = lcopy(3, x_ref.at[0, pl.ds(o2_, EIGHT), :], xk2)
        L4 = lcopy(4, resid_ref.at[pl.ds(o1_, EIGHT), :], rsd1)
        L5 = lcopy(5, resid_ref.at[pl.ds(o2_, EIGHT), :], rsd2)
        L6 = lcopy(6, g_ref, gv)

        barrier_sem = pltpu.get_barrier_semaphore()
        for nbr in (q1, q2):
            pl.semaphore_signal(
                barrier_sem, inc=1,
                device_id=(nbr,), device_id_type=pl.DeviceIdType.MESH,
            )
        pl.semaphore_wait(barrier_sem, 2)

        o1, o2, f1, f2 = o1_, o2_, f1_, f2_
        SIX = EIGHT // 2

        s1a1 = xchg(0, x_ref.at[0, pl.ds((QUART - a1) + f1, EIGHT), :],
                    r1.at[pl.ds(f1, EIGHT), :], q1)
        s1a2 = xchg(1, x_ref.at[0, pl.ds((QUART - a1) + c1, EIGHT), :],
                    r1.at[pl.ds(c1, EIGHT), :], q1)
        s1b1 = xchg(2, x_ref.at[0, pl.ds(HALF + (QUART - a2) + c2, EIGHT), :],
                    r2.at[pl.ds(c2, EIGHT), :], q2)
        s1b2 = xchg(3, x_ref.at[0, pl.ds(HALF + (QUART - a2) + f2, EIGHT), :],
                    r2.at[pl.ds(f2, EIGHT), :], q2)

        s1a1.wait_recv()
        L0.wait()
        r1[pl.ds(f1, EIGHT), :] = (
            r1[pl.ds(f1, EIGHT), :] + xf1[:, :]
        )
        s2a1 = xchg(4, r1.at[pl.ds(f1, SIX), :], r3.at[pl.ds(0, SIX), :], q2)
        s2a2 = xchg(5, r1.at[pl.ds(f1 + SIX, SIX), :],
                    r3.at[pl.ds(SIX, SIX), :], q2)

        s1b1.wait_recv()
        L1.wait()
        r2[pl.ds(f2, EIGHT), :] = (
            r2[pl.ds(f2, EIGHT), :] + xf2[:, :]
        )
        s2b1 = xchg(6, r2.at[pl.ds(f2, SIX), :], r4.at[pl.ds(0, SIX), :], q1)
        s2b2 = xchg(7, r2.at[pl.ds(f2 + SIX, SIX), :],
                    r4.at[pl.ds(SIX, SIX), :], q1)

        s1a2.wait_recv()
        L2.wait()
        L4.wait()
        t1[:, :] = r1[pl.ds(c1, EIGHT), :] + xk1[:, :] + rsd1[:, :]
        s1b2.wait_recv()
        L3.wait()
        L5.wait()
        L6.wait()
        t2[:, :] = r2[pl.ds(c2, EIGHT), :] + xk2[:, :] + rsd2[:, :]

        def norm_store(rbuf, tbuf, j, dst_off):
            y = rbuf[pl.ds(j, SIX), :] + tbuf[pl.ds(j, SIX), :]
            inv = lax.rsqrt(jnp.mean(y * y, axis=1, keepdims=True) + 1e-6)
            ov[pl.ds(dst_off, SIX), :] = y * inv * gv[0, :]

        s2a1.wait_recv()
        norm_store(r3, t1, 0, o1)
        s3a1 = xchg(8, ov.at[pl.ds(o1, SIX), :],
                    ov.at[pl.ds(o1, SIX), :], q2)
        s2b1.wait_recv()
        norm_store(r4, t2, 0, o2)
        s3b1 = xchg(10, ov.at[pl.ds(o2, SIX), :],
                    ov.at[pl.ds(o2, SIX), :], q1)
        s2a2.wait_recv()
        norm_store(r3, t1, SIX, o1 + SIX)
        s3a2 = xchg(9, ov.at[pl.ds(o1 + SIX, SIX), :],
                    ov.at[pl.ds(o1 + SIX, SIX), :], q2)
        O1 = lcopy(7, ov.at[pl.ds(o1, EIGHT), :],
                   out_ref.at[pl.ds(o1, EIGHT), :])
        s2b2.wait_recv()
        norm_store(r4, t2, SIX, o2 + SIX)
        s3b2 = xchg(11, ov.at[pl.ds(o2 + SIX, SIX), :],
                    ov.at[pl.ds(o2 + SIX, SIX), :], q1)
        O2 = lcopy(8, ov.at[pl.ds(o2, EIGHT), :],
                   out_ref.at[pl.ds(o2, EIGHT), :])

        s4a1 = xchg(12, ov.at[pl.ds(o1, EIGHT), :],
                    ov.at[pl.ds(o1, EIGHT), :], q1)
        s4b1 = xchg(14, ov.at[pl.ds(o2, EIGHT), :],
                    ov.at[pl.ds(o2, EIGHT), :], q2)
        s3a1.wait_recv()
        s3a2.wait_recv()
        s4a2 = xchg(13, ov.at[pl.ds(a1 + f1, EIGHT), :],
                    ov.at[pl.ds(a1 + f1, EIGHT), :], q1)
        O3 = lcopy(9, ov.at[pl.ds(a1 + f1, EIGHT), :],
                   out_ref.at[pl.ds(a1 + f1, EIGHT), :])
        s3b1.wait_recv()
        s3b2.wait_recv()
        s4b2 = xchg(15, ov.at[pl.ds(HALF + a2 + f2, EIGHT), :],
                    ov.at[pl.ds(HALF + a2 + f2, EIGHT), :], q2)
        O4 = lcopy(10, ov.at[pl.ds(HALF + a2 + f2, EIGHT), :],
                   out_ref.at[pl.ds(HALF + a2 + f2, EIGHT), :])

        s4a1.wait_recv()
        s4a2.wait_recv()
        O5 = lcopy(11, ov.at[pl.ds(QUART - a1, QUART), :],
                   out_ref.at[pl.ds(QUART - a1, QUART), :])
        s4b1.wait_recv()
        s4b2.wait_recv()
        O6 = lcopy(12, ov.at[pl.ds(HALF + (QUART - a2), QUART), :],
                   out_ref.at[pl.ds(HALF + (QUART - a2), QUART), :])

        for cp in (O1, O2, O3, O4, O5, O6):
            cp.wait()

        for s in (s1a1, s1a2, s1b1, s1b2, s2a1, s2a2, s2b1, s2b2,
                  s3a1, s3a2, s3b1, s3b2, s4a1, s4a2, s4b1, s4b2):
            s.wait_send()

    return pl.pallas_call(
        body,
        out_shape=jax.ShapeDtypeStruct((M, D), jnp.float32),
        in_specs=[
            pl.BlockSpec(memory_space=pl.ANY),
            pl.BlockSpec(memory_space=pl.ANY),
            pl.BlockSpec(memory_space=pl.ANY),
        ],
        out_specs=pl.BlockSpec(memory_space=pl.ANY),
        scratch_shapes=[
            pltpu.VMEM((QUART, D), jnp.float32),
            pltpu.VMEM((QUART, D), jnp.float32),
            pltpu.VMEM((EIGHT, D), jnp.float32),
            pltpu.VMEM((EIGHT, D), jnp.float32),
            pltpu.VMEM((EIGHT, D), jnp.float32),
            pltpu.VMEM((EIGHT, D), jnp.float32),
            pltpu.VMEM((EIGHT, D), jnp.float32),
            pltpu.VMEM((EIGHT, D), jnp.float32),
            pltpu.VMEM((EIGHT, D), jnp.float32),
            pltpu.VMEM((EIGHT, D), jnp.float32),
            pltpu.VMEM((EIGHT, D), jnp.float32),
            pltpu.VMEM((EIGHT, D), jnp.float32),
            pltpu.VMEM((1, D), jnp.float32),
            pltpu.VMEM((M, D), jnp.float32),
            pltpu.SemaphoreType.DMA((16,)),
            pltpu.SemaphoreType.DMA((16,)),
            pltpu.SemaphoreType.DMA((13,)),
        ],
        compiler_params=pltpu.CompilerParams(collective_id=0),
    )(x, resid, g)


# device time: 45788 ns/iter; 1.0130x vs baseline; 1.0130x over previous
import jax
import jax.numpy as jnp
from jax import lax
from jax.experimental import pallas as pl
from jax.experimental.pallas import tpu as pltpu

N_DEV = 4
M = 1024
D = 1024
HALF = M // 2
QUART = HALF // 2
EIGHT = HALF // 4


def kernel(partial, resid, gamma):
    x = partial
    g = gamma.reshape(1, D)

    def body(x_ref, resid_ref, g_ref, out_ref, r1, r2, r3, r4, t1, t2,
             send_sems, recv_sems):
        p = lax.axis_index("i")
        q1 = p + 1 - 2 * (p % 2)
        q2 = 3 - p

        def xchg(idx, src, dst, partner):
            rdma = pltpu.make_async_remote_copy(
                src_ref=src, dst_ref=dst,
                send_sem=send_sems.at[idx], recv_sem=recv_sems.at[idx],
                device_id=(partner,), device_id_type=pl.DeviceIdType.MESH,
            )
            rdma.start()
            return rdma

        a1 = jnp.where((p == 0) | (p == 3), 0, QUART)
        c1 = jnp.where(p <= 1, 0, EIGHT)
        a2 = jnp.where(p <= 1, 0, QUART)
        c2 = jnp.where((p == 0) | (p == 2), 0, EIGHT)

        barrier_sem = pltpu.get_barrier_semaphore()
        for nbr in (q1, q2):
            pl.semaphore_signal(
                barrier_sem, inc=1,
                device_id=(nbr,), device_id_type=pl.DeviceIdType.MESH,
            )
        pl.semaphore_wait(barrier_sem, 2)

        o1 = a1 + c1
        o2 = HALF + a2 + c2
        f1 = EIGHT - c1
        f2 = EIGHT - c2
        SIX = EIGHT // 2

        s1a1 = xchg(0, x_ref.at[0, pl.ds((QUART - a1) + f1, EIGHT), :],
                    r1.at[pl.ds(f1, EIGHT), :], q1)
        s1a2 = xchg(1, x_ref.at[0, pl.ds((QUART - a1) + c1, EIGHT), :],
                    r1.at[pl.ds(c1, EIGHT), :], q1)
        s1b1 = xchg(2, x_ref.at[0, pl.ds(HALF + (QUART - a2) + c2, EIGHT), :],
                    r2.at[pl.ds(c2, EIGHT), :], q2)
        s1b2 = xchg(3, x_ref.at[0, pl.ds(HALF + (QUART - a2) + f2, EIGHT), :],
                    r2.at[pl.ds(f2, EIGHT), :], q2)

        s1a1.wait_recv()
        r1[pl.ds(f1, EIGHT), :] = (
            r1[pl.ds(f1, EIGHT), :] + x_ref[0, pl.ds(a1 + f1, EIGHT), :]
        )
        s2a1 = xchg(4, r1.at[pl.ds(f1, SIX), :], r3.at[pl.ds(0, SIX), :], q2)
        s2a2 = xchg(5, r1.at[pl.ds(f1 + SIX, SIX), :],
                    r3.at[pl.ds(SIX, SIX), :], q2)

        s1b1.wait_recv()
        r2[pl.ds(f2, EIGHT), :] = (
            r2[pl.ds(f2, EIGHT), :] + x_ref[0, pl.ds(HALF + a2 + f2, EIGHT), :]
        )
        s2b1 = xchg(6, r2.at[pl.ds(f2, SIX), :], r4.at[pl.ds(0, SIX), :], q1)
        s2b2 = xchg(7, r2.at[pl.ds(f2 + SIX, SIX), :],
                    r4.at[pl.ds(SIX, SIX), :], q1)

        s1a2.wait_recv()
        t1[:, :] = (
            r1[pl.ds(c1, EIGHT), :]
            + x_ref[0, pl.ds(o1, EIGHT), :]
            + resid_ref[pl.ds(o1, EIGHT), :]
        )
        s1b2.wait_recv()
        t2[:, :] = (
            r2[pl.ds(c2, EIGHT), :]
            + x_ref[0, pl.ds(o2, EIGHT), :]
            + resid_ref[pl.ds(o2, EIGHT), :]
        )

        def norm_store(rbuf, tbuf, j, dst_off):
            y = rbuf[pl.ds(j, SIX), :] + tbuf[pl.ds(j, SIX), :]
            inv = lax.rsqrt(jnp.mean(y * y, axis=1, keepdims=True) + 1e-6)
            out_ref[pl.ds(dst_off, SIX), :] = y * inv * g_ref[0, :]

        s2a1.wait_recv()
        norm_store(r3, t1, 0, o1)
        s3a1 = xchg(8, out_ref.at[pl.ds(o1, SIX), :],
                    out_ref.at[pl.ds(o1, SIX), :], q2)
        s2b1.wait_recv()
        norm_store(r4, t2, 0, o2)
        s3b1 = xchg(10, out_ref.at[pl.ds(o2, SIX), :],
                    out_ref.at[pl.ds(o2, SIX), :], q1)
        s2a2.wait_recv()
        norm_store(r3, t1, SIX, o1 + SIX)
        s3a2 = xchg(9, out_ref.at[pl.ds(o1 + SIX, SIX), :],
                    out_ref.at[pl.ds(o1 + SIX, SIX), :], q2)
        s2b2.wait_recv()
        norm_store(r4, t2, SIX, o2 + SIX)
        s3b2 = xchg(11, out_ref.at[pl.ds(o2 + SIX, SIX), :],
                    out_ref.at[pl.ds(o2 + SIX, SIX), :], q1)

        s4a1 = xchg(12, out_ref.at[pl.ds(o1, EIGHT), :],
                    out_ref.at[pl.ds(o1, EIGHT), :], q1)
        s4b1 = xchg(14, out_ref.at[pl.ds(o2, EIGHT), :],
                    out_ref.at[pl.ds(o2, EIGHT), :], q2)
        s3a1.wait_recv()
        s3a2.wait_recv()
        s4a2 = xchg(13, out_ref.at[pl.ds(a1 + f1, EIGHT), :],
                    out_ref.at[pl.ds(a1 + f1, EIGHT), :], q1)
        s3b1.wait_recv()
        s3b2.wait_recv()
        s4b2 = xchg(15, out_ref.at[pl.ds(HALF + a2 + f2, EIGHT), :],
                    out_ref.at[pl.ds(HALF + a2 + f2, EIGHT), :], q2)

        s4a1.wait_recv()
        s4a2.wait_recv()
        s4b1.wait_recv()
        s4b2.wait_recv()

        for s in (s1a1, s1a2, s1b1, s1b2, s2a1, s2a2, s2b1, s2b2,
                  s3a1, s3a2, s3b1, s3b2, s4a1, s4a2, s4b1, s4b2):
            s.wait_send()

    return pl.pallas_call(
        body,
        out_shape=jax.ShapeDtypeStruct((M, D), jnp.float32),
        in_specs=[
            pl.BlockSpec(memory_space=pltpu.VMEM),
            pl.BlockSpec(memory_space=pltpu.VMEM),
            pl.BlockSpec(memory_space=pltpu.VMEM),
        ],
        out_specs=pl.BlockSpec(memory_space=pltpu.VMEM),
        scratch_shapes=[
            pltpu.VMEM((QUART, D), jnp.float32),
            pltpu.VMEM((QUART, D), jnp.float32),
            pltpu.VMEM((EIGHT, D), jnp.float32),
            pltpu.VMEM((EIGHT, D), jnp.float32),
            pltpu.VMEM((EIGHT, D), jnp.float32),
            pltpu.VMEM((EIGHT, D), jnp.float32),
            pltpu.SemaphoreType.DMA((16,)),
            pltpu.SemaphoreType.DMA((16,)),
        ],
        compiler_params=pltpu.CompilerParams(collective_id=0),
    )(x, resid, g)
